# Initial kernel scaffold; baseline (speedup 1.0000x reference)
import jax
import jax.numpy as jnp
from jax.experimental import pallas as pl


def kernel(x):
    raise NotImplementedError("write your pallas kernel here")


# speedup vs baseline: 3.8495x; 3.8495x over previous
import jax
import jax.numpy as jnp
from jax import lax
from jax.experimental import pallas as pl

_K = 64
_ROWS_PER_BLOCK = 8
_LANES = 128


def _topk_mask_body(x_ref, o_ref):
    x = x_ref[...]
    r_dim, g_dim, l_dim = x.shape
    bits = lax.bitcast_convert_type(x, jnp.int32)
    s = bits ^ (lax.shift_right_arithmetic(bits, 31) & jnp.int32(0x7FFFFFFF))
    imin = jnp.int32(-2147483648)

    def count_ge(cand_s):
        m = (s >= cand_s).astype(jnp.int32)
        return jnp.sum(m, axis=(1, 2)).reshape(r_dim, 1, 1)

    def bit_step(i, pu):
        cand_u = pu | (jnp.int32(1) << (31 - i))
        cnt = count_ge(cand_u ^ imin)
        return jnp.where(cnt >= _K, cand_u, pu)

    pu = lax.fori_loop(0, 32, bit_step, jnp.zeros((r_dim, 1, 1), jnp.int32))
    t_s = pu ^ imin

    gt = s > t_s
    eq = s == t_s
    count_gt = jnp.sum(gt.astype(jnp.int32), axis=(1, 2)).reshape(r_dim, 1, 1)
    need = _K - count_gt

    idx = (lax.broadcasted_iota(jnp.int32, x.shape, 1) * l_dim
           + lax.broadcasted_iota(jnp.int32, x.shape, 2))

    def idx_step(i, carry):
        lo, hi = carry
        mid = (lo + hi) >> 1
        cnt = jnp.sum(jnp.where(eq & (idx <= mid), 1, 0),
                      axis=(1, 2)).reshape(r_dim, 1, 1)
        pred = cnt >= need
        return jnp.where(pred, lo, mid + 1), jnp.where(pred, mid, hi)

    n_bits = (g_dim * l_dim - 1).bit_length()
    lo, _ = lax.fori_loop(0, n_bits, idx_step,
                          (jnp.zeros((r_dim, 1, 1), jnp.int32),
                           jnp.full((r_dim, 1, 1), g_dim * l_dim - 1, jnp.int32)))

    keep = gt | (eq & (idx <= lo))
    o_ref[...] = jnp.where(keep, jnp.maximum(x, 0.0), jnp.float32(0.0))


def kernel(x):
    n_rows, n = x.shape
    g = n // _LANES
    xr = x.reshape(n_rows, g, _LANES)
    out = pl.pallas_call(
        _topk_mask_body,
        grid=(n_rows // _ROWS_PER_BLOCK,),
        in_specs=[pl.BlockSpec((_ROWS_PER_BLOCK, g, _LANES), lambda i: (i, 0, 0))],
        out_specs=pl.BlockSpec((_ROWS_PER_BLOCK, g, _LANES), lambda i: (i, 0, 0)),
        out_shape=jax.ShapeDtypeStruct((n_rows, g, _LANES), jnp.float32),
    )(xr)
    return out.reshape(n_rows, n)


# speedup vs baseline: 4.2143x; 1.0948x over previous
import jax
import jax.numpy as jnp
from jax import lax
from jax.experimental import pallas as pl

_K = 64
_ROWS_PER_BLOCK = 8
_LANES = 128


def _topk_mask_body(x_ref, o_ref):
    x = x_ref[...]
    r_dim, g_dim, l_dim = x.shape
    bits = lax.bitcast_convert_type(x, jnp.int32)
    s = bits ^ (lax.shift_right_arithmetic(bits, 31) & jnp.int32(0x7FFFFFFF))
    imin = jnp.int32(-2147483648)

    def count_ge(cand_s):
        m = (s >= cand_s).astype(jnp.int32)
        return jnp.sum(m, axis=(1, 2)).reshape(r_dim, 1, 1)

    def bit_step(i, pu):
        cand_u = pu | (jnp.int32(1) << (31 - i))
        cnt = count_ge(cand_u ^ imin)
        return jnp.where(cnt >= _K, cand_u, pu)

    pu = lax.fori_loop(0, 32, bit_step, jnp.zeros((r_dim, 1, 1), jnp.int32))
    t_s = pu ^ imin

    gt = s > t_s
    eq = s == t_s
    count_gt = jnp.sum(gt.astype(jnp.int32), axis=(1, 2)).reshape(r_dim, 1, 1)
    need = _K - count_gt

    gidx = lax.broadcasted_iota(jnp.int32, x.shape, 1)

    def g_step(i, carry):
        lo, hi = carry
        mid = (lo + hi) >> 1
        cnt = jnp.sum(jnp.where(eq & (gidx <= mid), 1, 0),
                      axis=(1, 2)).reshape(r_dim, 1, 1)
        pred = cnt >= need
        return jnp.where(pred, lo, mid + 1), jnp.where(pred, mid, hi)

    g_bits = (g_dim - 1).bit_length()
    h, _ = lax.fori_loop(0, g_bits, g_step,
                         (jnp.zeros((r_dim, 1, 1), jnp.int32),
                          jnp.full((r_dim, 1, 1), g_dim - 1, jnp.int32)))
    cnt_before = jnp.sum(jnp.where(eq & (gidx < h), 1, 0),
                         axis=(1, 2)).reshape(r_dim, 1, 1)
    need2 = (need - cnt_before).astype(jnp.float32)

    tri_l = (lax.broadcasted_iota(jnp.int32, (l_dim, l_dim), 0)
             <= lax.broadcasted_iota(jnp.int32, (l_dim, l_dim), 1)
             ).astype(jnp.float32)
    eq2 = eq.astype(jnp.float32).reshape(r_dim * g_dim, l_dim)
    lane_pfx = jnp.dot(eq2, tri_l,
                       preferred_element_type=jnp.float32
                       ).reshape(r_dim, g_dim, l_dim)

    keep_eq = eq & ((gidx < h) | ((gidx == h) & (lane_pfx <= need2)))
    keep = gt | keep_eq
    o_ref[...] = jnp.where(keep, jnp.maximum(x, 0.0), jnp.float32(0.0))


def kernel(x):
    n_rows, n = x.shape
    g = n // _LANES
    xr = x.reshape(n_rows, g, _LANES)
    out = pl.pallas_call(
        _topk_mask_body,
        grid=(n_rows // _ROWS_PER_BLOCK,),
        in_specs=[pl.BlockSpec((_ROWS_PER_BLOCK, g, _LANES), lambda i: (i, 0, 0))],
        out_specs=pl.BlockSpec((_ROWS_PER_BLOCK, g, _LANES), lambda i: (i, 0, 0)),
        out_shape=jax.ShapeDtypeStruct((n_rows, g, _LANES), jnp.float32),
    )(xr)
    return out.reshape(n_rows, n)
